# baseline (speedup 1.0000x reference)
import functools
import math

import jax
import jax.numpy as jnp
from jax import lax
from jax.experimental import pallas as pl
from jax.experimental.pallas import tpu as pltpu
from jax.experimental.pallas import tpu_sc as plsc

VOCAB = 1000000
D = 64
BATCH = 4096
HIST = 200
B = BATCH * HIST
SCALE = math.sqrt(D)

CHUNK = 800


@jax.jit
def _embed(x_flat, table):
    info = plsc.get_sparse_core_info()
    nw = info.num_cores * info.num_subcores
    b_per_w = B // nw
    n_chunks = b_per_w // CHUNK

    mesh = plsc.VectorSubcoreMesh(core_axis_name="c", subcore_axis_name="s")

    @functools.partial(
        pl.kernel,
        mesh=mesh,
        out_type=jax.ShapeDtypeStruct((B, D), jnp.float32),
        compiler_params=pltpu.CompilerParams(use_tc_tiling_on_sc=False),
        scratch_types=[
            pltpu.VMEM((CHUNK,), jnp.int32),
            pltpu.VMEM((CHUNK, D), jnp.float32),
            pltpu.SemaphoreType.DMA,
        ],
    )
    def k(x_hbm, table_hbm, out_hbm, idx_v, rows_v, sem):
        wid = lax.axis_index("s") * info.num_cores + lax.axis_index("c")
        wbase = wid * b_per_w

        def chunk_body(g, carry):
            base = pl.multiple_of(wbase + g * CHUNK, 8)
            pltpu.sync_copy(x_hbm.at[pl.ds(base, CHUNK)], idx_v)
            pltpu.async_copy(table_hbm.at[idx_v], rows_v, sem).wait()

            def mul_body(r, c2):
                for j in range(D // 16):
                    v = rows_v[r, pl.ds(j * 16, 16)]
                    rows_v[r, pl.ds(j * 16, 16)] = v * SCALE
                return c2

            lax.fori_loop(0, CHUNK, mul_body, 0, unroll=2)
            pltpu.sync_copy(rows_v, out_hbm.at[pl.ds(base, CHUNK)])
            return carry

        lax.fori_loop(0, n_chunks, chunk_body, 0)

    return k(x_flat, table)


def kernel(x, table):
    x_flat = x.reshape(-1).astype(jnp.int32)
    out = _embed(x_flat, table)
    return out.reshape(BATCH, HIST, D)


# speedup vs baseline: 1.0717x; 1.0717x over previous
import functools
import math

import jax
import jax.numpy as jnp
from jax import lax
from jax.experimental import pallas as pl
from jax.experimental.pallas import tpu as pltpu
from jax.experimental.pallas import tpu_sc as plsc

VOCAB = 1000000
D = 64
BATCH = 4096
HIST = 200
B = BATCH * HIST
SCALE = math.sqrt(D)

CHUNK = 400
NBUF = 2


@jax.jit
def _embed(x_flat, table):
    info = plsc.get_sparse_core_info()
    nw = info.num_cores * info.num_subcores
    b_per_w = B // nw
    n_chunks = b_per_w // CHUNK

    mesh = plsc.VectorSubcoreMesh(core_axis_name="c", subcore_axis_name="s")

    @functools.partial(
        pl.kernel,
        mesh=mesh,
        out_type=jax.ShapeDtypeStruct((B, D), jnp.float32),
        compiler_params=pltpu.CompilerParams(use_tc_tiling_on_sc=False),
        scratch_types=[
            pltpu.VMEM((b_per_w,), jnp.int32),
            pltpu.VMEM((CHUNK, D), jnp.float32),
            pltpu.VMEM((CHUNK, D), jnp.float32),
            pltpu.VMEM((CHUNK, D), jnp.float32),
            pltpu.VMEM((CHUNK, D), jnp.float32),
            pltpu.SemaphoreType.DMA,
            pltpu.SemaphoreType.DMA,
            pltpu.SemaphoreType.DMA,
            pltpu.SemaphoreType.DMA,
        ],
    )
    def k(x_hbm, table_hbm, out_hbm, idx_v, g0, g1, s0, s1,
          gsem0, gsem1, osem0, osem1):
        gbuf = (g0, g1)
        sbuf = (s0, s1)
        gsem = (gsem0, gsem1)
        osem = (osem0, osem1)

        wid = lax.axis_index("s") * info.num_cores + lax.axis_index("c")
        wbase = wid * b_per_w

        pltpu.sync_copy(x_hbm.at[pl.ds(pl.multiple_of(wbase, 8), b_per_w)],
                        idx_v)

        for s in range(NBUF):
            pltpu.async_copy(
                table_hbm.at[idx_v.at[pl.ds(s * CHUNK, CHUNK)]],
                gbuf[s], gsem[s])

        @pl.loop(0, n_chunks, step=NBUF)
        def outer(grp):
            for s in range(NBUF):
                cur = grp + s
                pltpu.make_async_copy(
                    table_hbm.at[idx_v.at[pl.ds(0, CHUNK)]],
                    gbuf[s], gsem[s]).wait()
                @pl.when(cur >= NBUF)
                def _():
                    pltpu.make_async_copy(
                        sbuf[s], out_hbm.at[pl.ds(0, CHUNK)],
                        osem[s]).wait()

                @plsc.parallel_loop(0, CHUNK, unroll=8)
                def mul(r):
                    for j in range(D // 16):
                        v = gbuf[s][r, pl.ds(16 * j, 16)]
                        sbuf[s][r, pl.ds(16 * j, 16)] = v * SCALE

                base = pl.multiple_of(wbase + cur * CHUNK, 8)
                pltpu.async_copy(
                    sbuf[s], out_hbm.at[pl.ds(base, CHUNK)], osem[s])

                @pl.when(cur + NBUF < n_chunks)
                def _():
                    nb = pl.multiple_of((cur + NBUF) * CHUNK, 8)
                    pltpu.async_copy(
                        table_hbm.at[idx_v.at[pl.ds(nb, CHUNK)]],
                        gbuf[s], gsem[s])

        for s in range(NBUF):
            pltpu.make_async_copy(
                sbuf[s], out_hbm.at[pl.ds(0, CHUNK)], osem[s]).wait()

    return k(x_flat, table)


def kernel(x, table):
    x_flat = x.reshape(-1).astype(jnp.int32)
    out = _embed(x_flat, table)
    return out.reshape(BATCH, HIST, D)
